# baseline (speedup 1.0000x reference)
import dataclasses
import functools

import jax
import jax.numpy as jnp
from jax import lax
from jax.experimental import pallas as pl
from jax.experimental.pallas import tpu as pltpu
from jax.experimental.pallas import tpu_sc as plsc

_RHO = 0.2
_TAU = 1.0
_LANES = 16
_FINITE = 2139095040


def _fused_body(nsteps, cpr, blk, x_ref, w_ref, a_ref, b_ref,
                s_ref, z_ref, ne_ref):
    i = pl.program_id(0)
    xb = x_ref[...].astype(jnp.bfloat16)
    wb = w_ref[...].astype(jnp.bfloat16)
    res = lax.dot_general(
        wb, xb, (((1,), (1,)), ((), ())),
        preferred_element_type=jnp.float32)
    r = i // cpr
    c = (i % cpr) * blk
    s_ref[pl.ds(r, 1), pl.ds(c, blk)] = res[0:1, :]

    @pl.when(i == nsteps - 1)
    def _():
        s_raw = s_ref[...]
        a = a_ref[...]
        bias = b_ref[0, 0]
        s = jnp.where(a == 0.0, jnp.float32(-1e9), s_raw * a + bias)
        u = s / _TAU
        m = jnp.max(u, axis=1, keepdims=True)
        e = jnp.exp(u - m)
        zden = jnp.sum(e, axis=1, keepdims=True)
        p = e / zden
        ent = -jnp.sum(p * jnp.log(p + 1e-12), axis=1, keepdims=True)
        asum = jnp.sum(a, axis=1, keepdims=True)
        t_eff = jnp.maximum(asum, 1.0)
        ne_ref[...] = jnp.full((1, 1), jnp.mean(ent / jnp.log(t_eff)),
                               jnp.float32)
        k = jnp.maximum(jnp.round(_RHO * asum), 1.0)
        z_ref[...] = k * p


def _sc_topk(T, s_hbm, a_hbm, h_hbm, srow, arow, krow, cbuf, hrow, sem):
    nv = T // _LANES
    wid = lax.axis_index("s") * 2 + lax.axis_index("c")
    nrows = s_hbm.shape[0]

    @pl.when(wid < nrows)
    def _():
        r = wid
        pltpu.async_copy(s_hbm.at[r], srow, sem).wait()
        pltpu.async_copy(a_hbm.at[r], arow, sem).wait()

        one = jnp.int32(1)
        zero = jnp.int32(0)

        def build(i, carry):
            asum_acc, pos_acc, minv, maxv = carry
            sl = pl.ds(i * _LANES, _LANES)
            sv = srow[sl]
            av = arow[sl]
            se = jnp.where(av == 0.0, jnp.float32(-1e30), sv * av)
            bits = lax.bitcast_convert_type(se, jnp.int32)
            key = jnp.where(bits >= 0, bits, bits ^ jnp.int32(0x7FFFFFFF))
            krow[sl] = key
            cbuf[sl] = key
            return (asum_acc + av,
                    pos_acc + jnp.where(key >= 0, one, zero),
                    jnp.minimum(minv, key),
                    jnp.maximum(maxv, key))

        asum_acc, pos_acc, minv, maxv = lax.fori_loop(
            0, nv, build,
            (jnp.zeros((_LANES,), jnp.float32),
             jnp.zeros((_LANES,), jnp.int32),
             jnp.full((_LANES,), _FINITE, jnp.int32),
             jnp.full((_LANES,), -_FINITE, jnp.int32)))
        asum = jnp.sum(asum_acc)
        c0 = jnp.sum(pos_acc)
        minkey = jnp.min(minv)
        maxkey = jnp.max(maxv)
        kf = jnp.float32(_RHO) * asum
        kc = kf.astype(jnp.int32)
        kd = kf - kc.astype(jnp.float32)
        kr = kc + jnp.where(kd > 0.5, one, zero) - jnp.where(kd < -0.5, one, zero)
        ki = jnp.maximum(kr, 1)

        cond0 = c0 >= ki
        lo0 = jnp.where(cond0, jnp.maximum(zero, minkey), minkey)
        hi0 = jnp.where(cond0, maxkey + 1, jnp.minimum(zero, maxkey + 1))

        iota = lax.iota(jnp.int32, _LANES)
        sentinel = jnp.full((_LANES,), jnp.int32(-2147483648), jnp.int32)

        def wcond(carry):
            lo, hi, ic, nc = carry
            return hi - lo > 1

        def wbody(carry):
            lo, hi, ic, nc = carry
            mid = lo + ((hi - lo) >> 1)
            nvec = (nc + _LANES - 1) >> 4
            midv = jnp.full((_LANES,), mid, jnp.int32)
            lov = jnp.full((_LANES,), lo, jnp.int32)
            hiv = jnp.full((_LANES,), hi, jnp.int32)

            def pbody(i, acc):
                cntv, incv, w = acc
                kv = cbuf[pl.ds(i * _LANES, _LANES)]
                ge_lo = kv >= lov
                lt_hi = kv < hiv
                ge_hi = kv >= hiv
                ge_mid = kv >= midv
                keep = ge_lo & lt_hi
                cntv = cntv + jnp.where(ge_mid, one, zero)
                incv = incv + jnp.where(ge_hi, one, zero)
                keep_i = jnp.where(keep, one, zero)
                pref = plsc.cumsum(keep_i)
                pos = pref + jnp.full((_LANES,), w - 1, jnp.int32)
                plsc.store_scatter(cbuf, [pos], kv, mask=keep)
                return (cntv, incv, w + jnp.max(pref))

            cntv, incv, w = lax.fori_loop(
                0, nvec, pbody,
                (jnp.zeros((_LANES,), jnp.int32),
                 jnp.zeros((_LANES,), jnp.int32), zero))
            plsc.store_scatter(cbuf, [iota + jnp.full((_LANES,), w, jnp.int32)],
                               sentinel)
            c_mid = ic + jnp.sum(cntv)
            take = c_mid >= ki
            lo2 = jnp.where(take, mid, lo)
            hi2 = jnp.where(take, hi, mid)
            return (lo2, hi2, ic + jnp.sum(incv), w)

        lo, _hi, _ic, _nc = lax.while_loop(
            wcond, wbody, (lo0, hi0, zero, jnp.int32(T)))

        lv = jnp.full((_LANES,), lo, jnp.int32)

        def mbody(i, carry):
            sl = pl.ds(i * _LANES, _LANES)
            kv = krow[sl]
            hrow[sl] = jnp.where(kv >= lv, jnp.float32(1.0), jnp.float32(0.0))
            return carry

        lax.fori_loop(0, nv, mbody, 0)
        pltpu.async_copy(hrow, h_hbm.at[r], sem).wait()


def kernel(embeddings, attn, W, b):
    B, T, D = embeddings.shape
    BT = B * T
    BLK = 1024

    x2 = embeddings.reshape(BT, D)
    w8 = jnp.broadcast_to(W, (8, D))
    b8 = jnp.broadcast_to(b.reshape(1, 1), (8, 128))

    nsteps = BT // BLK
    cpr = T // BLK
    s, z, ne = pl.pallas_call(
        functools.partial(_fused_body, nsteps, cpr, BLK),
        grid=(nsteps,),
        in_specs=[
            pl.BlockSpec((BLK, D), lambda i: (i, 0)),
            pl.BlockSpec((8, D), lambda i: (0, 0)),
            pl.BlockSpec((B, T), lambda i: (0, 0)),
            pl.BlockSpec((8, 128), lambda i: (0, 0)),
        ],
        out_specs=[
            pl.BlockSpec((B, T), lambda i: (0, 0)),
            pl.BlockSpec((B, T), lambda i: (0, 0)),
            pl.BlockSpec((1, 1), lambda i: (0, 0)),
        ],
        out_shape=[
            jax.ShapeDtypeStruct((B, T), jnp.float32),
            jax.ShapeDtypeStruct((B, T), jnp.float32),
            jax.ShapeDtypeStruct((1, 1), jnp.float32),
        ],
    )(x2, w8, attn, b8)

    mesh = plsc.VectorSubcoreMesh(core_axis_name="c", subcore_axis_name="s")
    cp = pltpu.CompilerParams()
    if "needs_layout_passes" in pltpu.CompilerParams.__dataclass_fields__:
        cp = dataclasses.replace(cp, needs_layout_passes=False)
    sc_fn = functools.partial(
        pl.kernel,
        mesh=mesh,
        compiler_params=cp,
        out_type=jax.ShapeDtypeStruct((B, T), jnp.float32),
        scratch_types=[
            pltpu.VMEM((T,), jnp.float32),
            pltpu.VMEM((T,), jnp.float32),
            pltpu.VMEM((T,), jnp.int32),
            pltpu.VMEM((T + _LANES,), jnp.int32),
            pltpu.VMEM((T,), jnp.float32),
            pltpu.SemaphoreType.DMA,
        ],
    )(functools.partial(_sc_topk, T))
    h = sc_fn(s, attn)

    return (h, z, ne[0, 0])


# speedup vs baseline: 1.0132x; 1.0132x over previous
import dataclasses
import functools

import jax
import jax.numpy as jnp
from jax import lax
from jax.experimental import pallas as pl
from jax.experimental.pallas import tpu as pltpu
from jax.experimental.pallas import tpu_sc as plsc

_RHO = 0.2
_TAU = 1.0
_LANES = 16
_FINITE = 2139095040


def _fused_body(nsteps, cpr, blk, x_ref, w_ref, a_ref, b_ref,
                s_ref, z_ref, ne_ref):
    i = pl.program_id(0)
    xb = x_ref[...].astype(jnp.bfloat16)
    wb = w_ref[...].astype(jnp.bfloat16)
    res = lax.dot_general(
        wb, xb, (((1,), (1,)), ((), ())),
        preferred_element_type=jnp.float32)
    r = i // cpr
    c = (i % cpr) * blk
    s_ref[pl.ds(r, 1), pl.ds(c, blk)] = res[0:1, :]

    @pl.when(i == nsteps - 1)
    def _():
        s_raw = s_ref[...]
        a = a_ref[...]
        bias = b_ref[0, 0]
        s = jnp.where(a == 0.0, jnp.float32(-1e9), s_raw * a + bias)
        u = s / _TAU
        m = jnp.max(u, axis=1, keepdims=True)
        e = jnp.exp(u - m)
        zden = jnp.sum(e, axis=1, keepdims=True)
        p = e / zden
        ent = -jnp.sum(p * jnp.log(p + 1e-12), axis=1, keepdims=True)
        asum = jnp.sum(a, axis=1, keepdims=True)
        t_eff = jnp.maximum(asum, 1.0)
        ne_ref[...] = jnp.full((1, 1), jnp.mean(ent / jnp.log(t_eff)),
                               jnp.float32)
        k = jnp.maximum(jnp.round(_RHO * asum), 1.0)
        z_ref[...] = k * p


def _sc_topk(T, s_hbm, a_hbm, h_hbm, srow, arow, krow, cbuf, hrow, sem):
    nv = T // _LANES
    wid = lax.axis_index("s") * 2 + lax.axis_index("c")
    nrows = s_hbm.shape[0]

    @pl.when(wid < nrows)
    def _():
        r = wid
        pltpu.async_copy(s_hbm.at[r], srow, sem).wait()
        pltpu.async_copy(a_hbm.at[r], arow, sem).wait()

        one = jnp.int32(1)
        zero = jnp.int32(0)

        def build(i, carry):
            asum_acc, pos_acc, minv, maxv = carry
            sl = pl.ds(i * _LANES, _LANES)
            sv = srow[sl]
            av = arow[sl]
            se = jnp.where(av == 0.0, jnp.float32(-1e30), sv * av)
            bits = lax.bitcast_convert_type(se, jnp.int32)
            key = jnp.where(bits >= 0, bits, bits ^ jnp.int32(0x7FFFFFFF))
            krow[sl] = key
            cbuf[sl] = key
            return (asum_acc + av,
                    pos_acc + jnp.where(key >= 0, one, zero),
                    jnp.minimum(minv, key),
                    jnp.maximum(maxv, key))

        asum_acc, pos_acc, minv, maxv = lax.fori_loop(
            0, nv, build,
            (jnp.zeros((_LANES,), jnp.float32),
             jnp.zeros((_LANES,), jnp.int32),
             jnp.full((_LANES,), _FINITE, jnp.int32),
             jnp.full((_LANES,), -_FINITE, jnp.int32)))
        asum = jnp.sum(asum_acc)
        c0 = jnp.sum(pos_acc)
        minkey = jnp.min(minv)
        maxkey = jnp.max(maxv)
        kf = jnp.float32(_RHO) * asum
        kc = kf.astype(jnp.int32)
        kd = kf - kc.astype(jnp.float32)
        kr = kc + jnp.where(kd > 0.5, one, zero) - jnp.where(kd < -0.5, one, zero)
        ki = jnp.maximum(kr, 1)

        cond0 = c0 >= ki
        lo0 = jnp.where(cond0, jnp.maximum(zero, minkey), minkey)
        hi0 = jnp.where(cond0, maxkey + 1, jnp.minimum(zero, maxkey + 1))

        iota = lax.iota(jnp.int32, _LANES)
        sentinel = jnp.full((_LANES,), jnp.int32(-2147483648), jnp.int32)

        def wcond(carry):
            lo, hi, ic, nc = carry
            return hi - lo > 1

        def wbody(carry):
            lo, hi, ic, nc = carry
            mid = lo + ((hi - lo) >> 1)
            nvec = (nc + _LANES - 1) >> 4
            midv = jnp.full((_LANES,), mid, jnp.int32)
            lov = jnp.full((_LANES,), lo, jnp.int32)
            hiv = jnp.full((_LANES,), hi, jnp.int32)

            def pbody(i, acc):
                cntv, incv, woffv = acc
                kv = cbuf[pl.ds(i * _LANES, _LANES)]
                ge_lo = kv >= lov
                lt_hi = kv < hiv
                ge_hi = kv >= hiv
                ge_mid = kv >= midv
                keep = ge_lo & lt_hi
                cntv = cntv + jnp.where(ge_mid, one, zero)
                incv = incv + jnp.where(ge_hi, one, zero)
                keep_i = jnp.where(keep, one, zero)
                pref = plsc.cumsum(keep_i)
                pos = pref + woffv
                plsc.store_scatter(cbuf, [pos], kv, mask=keep)
                nkeep = plsc.all_reduce_population_count(keep)
                return (cntv, incv, woffv + nkeep)

            minus1 = jnp.full((_LANES,), jnp.int32(-1), jnp.int32)
            cntv, incv, woffv = lax.fori_loop(
                0, nvec, pbody,
                (jnp.zeros((_LANES,), jnp.int32),
                 jnp.zeros((_LANES,), jnp.int32), minus1))
            w = jnp.max(woffv) + 1
            plsc.store_scatter(cbuf, [iota + jnp.full((_LANES,), w, jnp.int32)],
                               sentinel)
            c_mid = ic + jnp.sum(cntv)
            take = c_mid >= ki
            lo2 = jnp.where(take, mid, lo)
            hi2 = jnp.where(take, hi, mid)
            return (lo2, hi2, ic + jnp.sum(incv), w)

        lo, _hi, _ic, _nc = lax.while_loop(
            wcond, wbody, (lo0, hi0, zero, jnp.int32(T)))

        lv = jnp.full((_LANES,), lo, jnp.int32)

        def mbody(i, carry):
            sl = pl.ds(i * _LANES, _LANES)
            kv = krow[sl]
            hrow[sl] = jnp.where(kv >= lv, jnp.float32(1.0), jnp.float32(0.0))
            return carry

        lax.fori_loop(0, nv, mbody, 0)
        pltpu.async_copy(hrow, h_hbm.at[r], sem).wait()


def kernel(embeddings, attn, W, b):
    B, T, D = embeddings.shape
    BT = B * T
    BLK = 1024

    x2 = embeddings.reshape(BT, D)
    w8 = jnp.broadcast_to(W, (8, D))
    b8 = jnp.broadcast_to(b.reshape(1, 1), (8, 128))

    nsteps = BT // BLK
    cpr = T // BLK
    s, z, ne = pl.pallas_call(
        functools.partial(_fused_body, nsteps, cpr, BLK),
        grid=(nsteps,),
        in_specs=[
            pl.BlockSpec((BLK, D), lambda i: (i, 0)),
            pl.BlockSpec((8, D), lambda i: (0, 0)),
            pl.BlockSpec((B, T), lambda i: (0, 0)),
            pl.BlockSpec((8, 128), lambda i: (0, 0)),
        ],
        out_specs=[
            pl.BlockSpec((B, T), lambda i: (0, 0)),
            pl.BlockSpec((B, T), lambda i: (0, 0)),
            pl.BlockSpec((1, 1), lambda i: (0, 0)),
        ],
        out_shape=[
            jax.ShapeDtypeStruct((B, T), jnp.float32),
            jax.ShapeDtypeStruct((B, T), jnp.float32),
            jax.ShapeDtypeStruct((1, 1), jnp.float32),
        ],
    )(x2, w8, attn, b8)

    mesh = plsc.VectorSubcoreMesh(core_axis_name="c", subcore_axis_name="s")
    cp = pltpu.CompilerParams()
    if "needs_layout_passes" in pltpu.CompilerParams.__dataclass_fields__:
        cp = dataclasses.replace(cp, needs_layout_passes=False)
    sc_fn = functools.partial(
        pl.kernel,
        mesh=mesh,
        compiler_params=cp,
        out_type=jax.ShapeDtypeStruct((B, T), jnp.float32),
        scratch_types=[
            pltpu.VMEM((T,), jnp.float32),
            pltpu.VMEM((T,), jnp.float32),
            pltpu.VMEM((T,), jnp.int32),
            pltpu.VMEM((T + _LANES,), jnp.int32),
            pltpu.VMEM((T,), jnp.float32),
            pltpu.SemaphoreType.DMA,
        ],
    )(functools.partial(_sc_topk, T))
    h = sc_fn(s, attn)

    return (h, z, ne[0, 0])


# speedup vs baseline: 1.0754x; 1.0614x over previous
import dataclasses
import functools

import jax
import jax.numpy as jnp
from jax import lax
from jax.experimental import pallas as pl
from jax.experimental.pallas import tpu as pltpu
from jax.experimental.pallas import tpu_sc as plsc

_RHO = 0.2
_TAU = 1.0
_LANES = 16
_FINITE = 2139095040


def _fused_body(nsteps, cpr, blk, x_ref, w_ref, a_ref, b_ref,
                s_ref, z_ref, ne_ref):
    i = pl.program_id(0)
    xb = x_ref[...].astype(jnp.bfloat16)
    wb = w_ref[...].astype(jnp.bfloat16)
    res = lax.dot_general(
        wb, xb, (((1,), (1,)), ((), ())),
        preferred_element_type=jnp.float32)
    r = i // cpr
    c = (i % cpr) * blk
    s_ref[pl.ds(r, 1), pl.ds(c, blk)] = res[0:1, :]

    @pl.when(i == nsteps - 1)
    def _():
        s_raw = s_ref[...]
        a = a_ref[...]
        bias = b_ref[0, 0]
        s = jnp.where(a == 0.0, jnp.float32(-1e9), s_raw * a + bias)
        u = s / _TAU
        m = jnp.max(u, axis=1, keepdims=True)
        e = jnp.exp(u - m)
        zden = jnp.sum(e, axis=1, keepdims=True)
        p = e / zden
        ent = -jnp.sum(p * jnp.log(p + 1e-12), axis=1, keepdims=True)
        asum = jnp.sum(a, axis=1, keepdims=True)
        t_eff = jnp.maximum(asum, 1.0)
        ne_ref[...] = jnp.full((1, 1), jnp.mean(ent / jnp.log(t_eff)),
                               jnp.float32)
        k = jnp.maximum(jnp.round(_RHO * asum), 1.0)
        z_ref[...] = k * p


def _sc_topk(T, s_hbm, a_hbm, h_hbm, srow, arow, krow, cbuf, hrow, sem):
    nv = T // _LANES
    wid = lax.axis_index("s") * 2 + lax.axis_index("c")
    nrows = s_hbm.shape[0]

    @pl.when(wid < nrows)
    def _():
        r = wid
        pltpu.async_copy(s_hbm.at[r], srow, sem).wait()
        pltpu.async_copy(a_hbm.at[r], arow, sem).wait()

        one = jnp.int32(1)
        zero = jnp.int32(0)

        def build(i, carry):
            asum_acc, pos_acc, minv, maxv = carry
            sl = pl.ds(i * _LANES, _LANES)
            sv = srow[sl]
            av = arow[sl]
            se = jnp.where(av == 0.0, jnp.float32(-1e30), sv * av)
            bits = lax.bitcast_convert_type(se, jnp.int32)
            key = jnp.where(bits >= 0, bits, bits ^ jnp.int32(0x7FFFFFFF))
            krow[sl] = key
            cbuf[sl] = key
            return (asum_acc + av,
                    pos_acc + jnp.where(key >= 0, one, zero),
                    jnp.minimum(minv, key),
                    jnp.maximum(maxv, key))

        asum_acc, pos_acc, minv, maxv = lax.fori_loop(
            0, nv, build,
            (jnp.zeros((_LANES,), jnp.float32),
             jnp.zeros((_LANES,), jnp.int32),
             jnp.full((_LANES,), _FINITE, jnp.int32),
             jnp.full((_LANES,), -_FINITE, jnp.int32)))
        asum = jnp.sum(asum_acc)
        c0 = jnp.sum(pos_acc)
        minkey = jnp.min(minv)
        maxkey = jnp.max(maxv)
        kf = jnp.float32(_RHO) * asum
        kc = kf.astype(jnp.int32)
        kd = kf - kc.astype(jnp.float32)
        kr = kc + jnp.where(kd > 0.5, one, zero) - jnp.where(kd < -0.5, one, zero)
        ki = jnp.maximum(kr, 1)

        cond0 = c0 >= ki
        lo0 = jnp.where(cond0, jnp.maximum(zero, minkey), minkey)
        hi0 = jnp.where(cond0, maxkey + 1, jnp.minimum(zero, maxkey + 1))

        iota = lax.iota(jnp.int32, _LANES)
        sentinel = jnp.full((_LANES,), jnp.int32(-2147483648), jnp.int32)

        def wcond(carry):
            lo, hi, ic, nc = carry
            return hi - lo > 1

        def wbody(carry):
            lo, hi, ic, nc = carry
            mid = lo + ((hi - lo) >> 1)
            npair = (nc + 2 * _LANES - 1) >> 5
            midv = jnp.full((_LANES,), mid, jnp.int32)
            lov = jnp.full((_LANES,), lo, jnp.int32)
            hiv = jnp.full((_LANES,), hi, jnp.int32)

            def pbody(i, acc):
                cntv, incv, woffv = acc
                kv0 = cbuf[pl.ds(i * 2 * _LANES, _LANES)]
                kv1 = cbuf[pl.ds(i * 2 * _LANES + _LANES, _LANES)]
                keep0 = (kv0 >= lov) & (kv0 < hiv)
                keep1 = (kv1 >= lov) & (kv1 < hiv)
                cntv = cntv + jnp.where(kv0 >= midv, one, zero) \
                            + jnp.where(kv1 >= midv, one, zero)
                incv = incv + jnp.where(kv0 >= hiv, one, zero) \
                            + jnp.where(kv1 >= hiv, one, zero)
                pref0 = plsc.cumsum(jnp.where(keep0, one, zero))
                pref1 = plsc.cumsum(jnp.where(keep1, one, zero))
                nk0 = plsc.all_reduce_population_count(keep0)
                nk1 = plsc.all_reduce_population_count(keep1)
                plsc.store_scatter(cbuf, [pref0 + woffv], kv0, mask=keep0)
                plsc.store_scatter(cbuf, [pref1 + woffv + nk0], kv1,
                                   mask=keep1)
                return (cntv, incv, woffv + nk0 + nk1)

            minus1 = jnp.full((_LANES,), jnp.int32(-1), jnp.int32)
            cntv, incv, woffv = lax.fori_loop(
                0, npair, pbody,
                (jnp.zeros((_LANES,), jnp.int32),
                 jnp.zeros((_LANES,), jnp.int32), minus1))
            w = jnp.max(woffv) + 1
            wv = jnp.full((_LANES,), w, jnp.int32)
            plsc.store_scatter(cbuf, [iota + wv], sentinel)
            plsc.store_scatter(cbuf, [iota + wv + jnp.full((_LANES,), _LANES,
                                                           jnp.int32)],
                               sentinel)
            c_mid = ic + jnp.sum(cntv)
            take = c_mid >= ki
            lo2 = jnp.where(take, mid, lo)
            hi2 = jnp.where(take, hi, mid)
            return (lo2, hi2, ic + jnp.sum(incv), w)

        lo, _hi, _ic, _nc = lax.while_loop(
            wcond, wbody, (lo0, hi0, zero, jnp.int32(T)))

        lv = jnp.full((_LANES,), lo, jnp.int32)

        def mbody(i, carry):
            sl = pl.ds(i * _LANES, _LANES)
            kv = krow[sl]
            hrow[sl] = jnp.where(kv >= lv, jnp.float32(1.0), jnp.float32(0.0))
            return carry

        lax.fori_loop(0, nv, mbody, 0)
        pltpu.async_copy(hrow, h_hbm.at[r], sem).wait()


def kernel(embeddings, attn, W, b):
    B, T, D = embeddings.shape
    BT = B * T
    BLK = 1024

    x2 = embeddings.reshape(BT, D)
    w8 = jnp.broadcast_to(W, (8, D))
    b8 = jnp.broadcast_to(b.reshape(1, 1), (8, 128))

    nsteps = BT // BLK
    cpr = T // BLK
    s, z, ne = pl.pallas_call(
        functools.partial(_fused_body, nsteps, cpr, BLK),
        grid=(nsteps,),
        in_specs=[
            pl.BlockSpec((BLK, D), lambda i: (i, 0)),
            pl.BlockSpec((8, D), lambda i: (0, 0)),
            pl.BlockSpec((B, T), lambda i: (0, 0)),
            pl.BlockSpec((8, 128), lambda i: (0, 0)),
        ],
        out_specs=[
            pl.BlockSpec((B, T), lambda i: (0, 0)),
            pl.BlockSpec((B, T), lambda i: (0, 0)),
            pl.BlockSpec((1, 1), lambda i: (0, 0)),
        ],
        out_shape=[
            jax.ShapeDtypeStruct((B, T), jnp.float32),
            jax.ShapeDtypeStruct((B, T), jnp.float32),
            jax.ShapeDtypeStruct((1, 1), jnp.float32),
        ],
    )(x2, w8, attn, b8)

    mesh = plsc.VectorSubcoreMesh(core_axis_name="c", subcore_axis_name="s")
    cp = pltpu.CompilerParams()
    if "needs_layout_passes" in pltpu.CompilerParams.__dataclass_fields__:
        cp = dataclasses.replace(cp, needs_layout_passes=False)
    sc_fn = functools.partial(
        pl.kernel,
        mesh=mesh,
        compiler_params=cp,
        out_type=jax.ShapeDtypeStruct((B, T), jnp.float32),
        scratch_types=[
            pltpu.VMEM((T,), jnp.float32),
            pltpu.VMEM((T,), jnp.float32),
            pltpu.VMEM((T,), jnp.int32),
            pltpu.VMEM((T + 2 * _LANES,), jnp.int32),
            pltpu.VMEM((T,), jnp.float32),
            pltpu.SemaphoreType.DMA,
        ],
    )(functools.partial(_sc_topk, T))
    h = sc_fn(s, attn)

    return (h, z, ne[0, 0])


# speedup vs baseline: 1.1095x; 1.0317x over previous
import dataclasses
import functools

import jax
import jax.numpy as jnp
from jax import lax
from jax.experimental import pallas as pl
from jax.experimental.pallas import tpu as pltpu
from jax.experimental.pallas import tpu_sc as plsc

_RHO = 0.2
_TAU = 1.0
_LANES = 16
_FINITE = 2139095040


def _fused_body(nsteps, cpr, blk, x_ref, w_ref, a_ref, b_ref,
                s_ref, z_ref, ne_ref):
    i = pl.program_id(0)
    xb = x_ref[...].astype(jnp.bfloat16)
    wb = w_ref[...].astype(jnp.bfloat16)
    res = lax.dot_general(
        wb, xb, (((1,), (1,)), ((), ())),
        preferred_element_type=jnp.float32)
    r = i // cpr
    c = (i % cpr) * blk
    s_ref[pl.ds(r, 1), pl.ds(c, blk)] = res[0:1, :]

    @pl.when(i == nsteps - 1)
    def _():
        s_raw = s_ref[...]
        a = a_ref[...]
        bias = b_ref[0, 0]
        s = jnp.where(a == 0.0, jnp.float32(-1e9), s_raw * a + bias)
        u = s / _TAU
        m = jnp.max(u, axis=1, keepdims=True)
        e = jnp.exp(u - m)
        zden = jnp.sum(e, axis=1, keepdims=True)
        p = e / zden
        ent = -jnp.sum(p * jnp.log(p + 1e-12), axis=1, keepdims=True)
        asum = jnp.sum(a, axis=1, keepdims=True)
        t_eff = jnp.maximum(asum, 1.0)
        ne_ref[...] = jnp.full((1, 1), jnp.mean(ent / jnp.log(t_eff)),
                               jnp.float32)
        k = jnp.maximum(jnp.round(_RHO * asum), 1.0)
        z_ref[...] = k * p


def _sc_topk(T, s_hbm, a_hbm, h_hbm, srow, arow, krow, cbuf, hrow, sem):
    nv = T // _LANES
    wid = lax.axis_index("s") * 2 + lax.axis_index("c")
    nrows = s_hbm.shape[0]

    @pl.when(wid < nrows)
    def _():
        r = wid
        pltpu.async_copy(s_hbm.at[r], srow, sem).wait()
        pltpu.async_copy(a_hbm.at[r], arow, sem).wait()

        one = jnp.int32(1)
        zero = jnp.int32(0)

        def build(i, carry):
            asum_acc, pos_acc, minv, maxv = carry
            sl = pl.ds(i * _LANES, _LANES)
            sv = srow[sl]
            av = arow[sl]
            se = jnp.where(av == 0.0, jnp.float32(-1e30), sv * av)
            bits = lax.bitcast_convert_type(se, jnp.int32)
            key = jnp.where(bits >= 0, bits, bits ^ jnp.int32(0x7FFFFFFF))
            krow[sl] = key
            cbuf[sl] = key
            return (asum_acc + av,
                    pos_acc + jnp.where(key >= 0, one, zero),
                    jnp.minimum(minv, key),
                    jnp.maximum(maxv, key))

        asum_acc, pos_acc, minv, maxv = lax.fori_loop(
            0, nv, build,
            (jnp.zeros((_LANES,), jnp.float32),
             jnp.zeros((_LANES,), jnp.int32),
             jnp.full((_LANES,), _FINITE, jnp.int32),
             jnp.full((_LANES,), -_FINITE, jnp.int32)))
        asum = jnp.sum(asum_acc)
        c0 = jnp.sum(pos_acc)
        minkey = jnp.min(minv)
        maxkey = jnp.max(maxv)
        kf = jnp.float32(_RHO) * asum
        kc = kf.astype(jnp.int32)
        kd = kf - kc.astype(jnp.float32)
        kr = kc + jnp.where(kd > 0.5, one, zero) - jnp.where(kd < -0.5, one, zero)
        ki = jnp.maximum(kr, 1)

        cond0 = c0 >= ki
        lo0 = jnp.where(cond0, jnp.maximum(zero, minkey), minkey)
        hi0 = jnp.where(cond0, maxkey + 1, jnp.minimum(zero, maxkey + 1))

        iota = lax.iota(jnp.int32, _LANES)
        sentinel = jnp.full((_LANES,), jnp.int32(-2147483648), jnp.int32)

        def wcond(carry):
            lo, hi, ic, nc = carry
            return hi - lo > 1

        def wbody(carry):
            lo, hi, ic, nc = carry
            mid = lo + ((hi - lo) >> 1)
            nquad = (nc + 4 * _LANES - 1) >> 6
            midv = jnp.full((_LANES,), mid, jnp.int32)
            lov = jnp.full((_LANES,), lo, jnp.int32)
            hiv = jnp.full((_LANES,), hi, jnp.int32)

            def pbody(i, acc):
                cntv, incv, woffv = acc
                base = i * 4 * _LANES
                kv0 = cbuf[pl.ds(base, _LANES)]
                kv1 = cbuf[pl.ds(base + _LANES, _LANES)]
                kv2 = cbuf[pl.ds(base + 2 * _LANES, _LANES)]
                kv3 = cbuf[pl.ds(base + 3 * _LANES, _LANES)]
                keep0 = (kv0 >= lov) & (kv0 < hiv)
                keep1 = (kv1 >= lov) & (kv1 < hiv)
                keep2 = (kv2 >= lov) & (kv2 < hiv)
                keep3 = (kv3 >= lov) & (kv3 < hiv)
                cntv = cntv + jnp.where(kv0 >= midv, one, zero) \
                            + jnp.where(kv1 >= midv, one, zero) \
                            + jnp.where(kv2 >= midv, one, zero) \
                            + jnp.where(kv3 >= midv, one, zero)
                incv = incv + jnp.where(kv0 >= hiv, one, zero) \
                            + jnp.where(kv1 >= hiv, one, zero) \
                            + jnp.where(kv2 >= hiv, one, zero) \
                            + jnp.where(kv3 >= hiv, one, zero)
                pref0 = plsc.cumsum(jnp.where(keep0, one, zero))
                pref1 = plsc.cumsum(jnp.where(keep1, one, zero))
                pref2 = plsc.cumsum(jnp.where(keep2, one, zero))
                pref3 = plsc.cumsum(jnp.where(keep3, one, zero))
                nk0 = plsc.all_reduce_population_count(keep0)
                nk1 = plsc.all_reduce_population_count(keep1)
                nk2 = plsc.all_reduce_population_count(keep2)
                nk3 = plsc.all_reduce_population_count(keep3)
                o1 = woffv + nk0
                o2 = o1 + nk1
                o3 = o2 + nk2
                plsc.store_scatter(cbuf, [pref0 + woffv], kv0, mask=keep0)
                plsc.store_scatter(cbuf, [pref1 + o1], kv1, mask=keep1)
                plsc.store_scatter(cbuf, [pref2 + o2], kv2, mask=keep2)
                plsc.store_scatter(cbuf, [pref3 + o3], kv3, mask=keep3)
                return (cntv, incv, o3 + nk3)

            minus1 = jnp.full((_LANES,), jnp.int32(-1), jnp.int32)
            cntv, incv, woffv = lax.fori_loop(
                0, nquad, pbody,
                (jnp.zeros((_LANES,), jnp.int32),
                 jnp.zeros((_LANES,), jnp.int32), minus1))
            w = jnp.max(woffv) + 1
            wv = jnp.full((_LANES,), w, jnp.int32)
            lanesv = jnp.full((_LANES,), _LANES, jnp.int32)
            plsc.store_scatter(cbuf, [iota + wv], sentinel)
            plsc.store_scatter(cbuf, [iota + wv + lanesv], sentinel)
            plsc.store_scatter(cbuf, [iota + wv + lanesv + lanesv], sentinel)
            plsc.store_scatter(cbuf, [iota + wv + lanesv + lanesv + lanesv],
                               sentinel)
            c_mid = ic + jnp.sum(cntv)
            take = c_mid >= ki
            lo2 = jnp.where(take, mid, lo)
            hi2 = jnp.where(take, hi, mid)
            return (lo2, hi2, ic + jnp.sum(incv), w)

        lo, _hi, _ic, _nc = lax.while_loop(
            wcond, wbody, (lo0, hi0, zero, jnp.int32(T)))

        lv = jnp.full((_LANES,), lo, jnp.int32)

        def mbody(i, carry):
            sl = pl.ds(i * _LANES, _LANES)
            kv = krow[sl]
            hrow[sl] = jnp.where(kv >= lv, jnp.float32(1.0), jnp.float32(0.0))
            return carry

        lax.fori_loop(0, nv, mbody, 0)
        pltpu.async_copy(hrow, h_hbm.at[r], sem).wait()


def kernel(embeddings, attn, W, b):
    B, T, D = embeddings.shape
    BT = B * T
    BLK = 1024

    x2 = embeddings.reshape(BT, D)
    w8 = jnp.broadcast_to(W, (8, D))
    b8 = jnp.broadcast_to(b.reshape(1, 1), (8, 128))

    nsteps = BT // BLK
    cpr = T // BLK
    s, z, ne = pl.pallas_call(
        functools.partial(_fused_body, nsteps, cpr, BLK),
        grid=(nsteps,),
        in_specs=[
            pl.BlockSpec((BLK, D), lambda i: (i, 0)),
            pl.BlockSpec((8, D), lambda i: (0, 0)),
            pl.BlockSpec((B, T), lambda i: (0, 0)),
            pl.BlockSpec((8, 128), lambda i: (0, 0)),
        ],
        out_specs=[
            pl.BlockSpec((B, T), lambda i: (0, 0)),
            pl.BlockSpec((B, T), lambda i: (0, 0)),
            pl.BlockSpec((1, 1), lambda i: (0, 0)),
        ],
        out_shape=[
            jax.ShapeDtypeStruct((B, T), jnp.float32),
            jax.ShapeDtypeStruct((B, T), jnp.float32),
            jax.ShapeDtypeStruct((1, 1), jnp.float32),
        ],
    )(x2, w8, attn, b8)

    mesh = plsc.VectorSubcoreMesh(core_axis_name="c", subcore_axis_name="s")
    cp = pltpu.CompilerParams()
    if "needs_layout_passes" in pltpu.CompilerParams.__dataclass_fields__:
        cp = dataclasses.replace(cp, needs_layout_passes=False)
    sc_fn = functools.partial(
        pl.kernel,
        mesh=mesh,
        compiler_params=cp,
        out_type=jax.ShapeDtypeStruct((B, T), jnp.float32),
        scratch_types=[
            pltpu.VMEM((T,), jnp.float32),
            pltpu.VMEM((T,), jnp.float32),
            pltpu.VMEM((T,), jnp.int32),
            pltpu.VMEM((T + 4 * _LANES,), jnp.int32),
            pltpu.VMEM((T,), jnp.float32),
            pltpu.SemaphoreType.DMA,
        ],
    )(functools.partial(_sc_topk, T))
    h = sc_fn(s, attn)

    return (h, z, ne[0, 0])
